# baseline (speedup 1.0000x reference)
import functools

import jax
import jax.numpy as jnp
from jax import lax
from jax.experimental import pallas as pl
from jax.experimental.pallas import tpu as pltpu
from jax.experimental.pallas import tpu_sc as plsc

_N_USERS = 1000000
_N_ITEMS = 100000
_N_OCC = 1000
_EMBED_DIM = 32
_B = 1024
_LANE = 128

_OFF_O = _N_USERS + _N_ITEMS
_OFF_T = _N_USERS + _N_ITEMS + _N_OCC
_OFF_I = _N_USERS


def _load_my_indices(u_hbm, o_hbm, t_hbm, i_hbm, idx_v, wid, per_w, nl):
  src = wid // 8
  start = (wid % 8) * per_w

  @pl.when(src == 0)
  def _():
    pltpu.sync_copy(u_hbm.at[pl.ds(start, per_w)], idx_v)

  @pl.when(src == 1)
  def _():
    pltpu.sync_copy(o_hbm.at[pl.ds(start, per_w)], idx_v)

  @pl.when(src == 2)
  def _():
    pltpu.sync_copy(t_hbm.at[pl.ds(start, per_w)], idx_v)

  @pl.when(src == 3)
  def _():
    pltpu.sync_copy(i_hbm.at[pl.ds(start, per_w)], idx_v)

  off = (jnp.where(src == 1, _OFF_O, 0)
         + jnp.where(src == 2, _OFF_T, 0)
         + jnp.where(src == 3, _OFF_I, 0)).astype(jnp.int32)

  def add_off(c, _):
    idx_v[pl.ds(c * nl, nl)] = idx_v[pl.ds(c * nl, nl)] + off
    return 0

  lax.fori_loop(0, per_w // nl, add_off, 0)


def _sc_gather_rows(table_t, u, o, t, i):
  info = plsc.get_sparse_core_info()
  nw = info.num_cores * info.num_subcores
  nl = info.num_lanes
  n = 4 * _B
  per_w = n // nw
  nchunk = per_w // nl

  mesh = plsc.VectorSubcoreMesh(core_axis_name="c", subcore_axis_name="s")

  nh = nl // 2
  nhc = per_w // nh

  @functools.partial(
      pl.kernel,
      out_type=jax.ShapeDtypeStruct((_EMBED_DIM, n), jnp.float32),
      mesh=mesh,
      compiler_params=pltpu.CompilerParams(needs_layout_passes=False),
      scratch_types=[
          pltpu.VMEM((per_w + nh,), jnp.int32),
          pltpu.VMEM((2, nh, _EMBED_DIM, _LANE), jnp.float32),
          pltpu.VMEM((_EMBED_DIM, per_w + nh), jnp.float32),
          pltpu.SemaphoreType.DMA,
          pltpu.SemaphoreType.DMA,
      ],
  )
  def k(table_hbm, u_hbm, o_hbm, t_hbm, i_hbm, rows_out,
        idx_v, blk_v, rows_v, sem_0, sem_1):
    wid = lax.axis_index("s") * info.num_cores + lax.axis_index("c")
    base = wid * per_w
    _load_my_indices(u_hbm, o_hbm, t_hbm, i_hbm,
                     idx_v.at[pl.ds(0, per_w)], wid, per_w, nl)

    lanes = lax.iota(jnp.int32, nl)

    def issue(hc):
      buf = hc % 2
      chunk = idx_v[pl.ds(hc * nh, nl)]

      def one(j, _):
        idx = jnp.sum(jnp.where(lanes == j, chunk, 0))
        blk = pl.multiple_of((idx // _LANE) * _LANE, _LANE)

        @pl.when(buf == 0)
        def _():
          pltpu.async_copy(table_hbm.at[:, pl.ds(blk, _LANE)],
                           blk_v.at[0, j], sem_0)

        @pl.when(buf == 1)
        def _():
          pltpu.async_copy(table_hbm.at[:, pl.ds(blk, _LANE)],
                           blk_v.at[1, j], sem_1)

        return 0

      lax.fori_loop(0, nh, one, 0)

    def drain(hc):
      def one(j, _):
        @pl.when(hc % 2 == 0)
        def _():
          pltpu.make_async_copy(table_hbm.at[:, pl.ds(0, _LANE)],
                                blk_v.at[0, 0], sem_0).wait()

        @pl.when(hc % 2 == 1)
        def _():
          pltpu.make_async_copy(table_hbm.at[:, pl.ds(0, _LANE)],
                                blk_v.at[1, 0], sem_1).wait()

        return 0

      lax.fori_loop(0, nh, one, 0)

    issue(0)

    def hc_body(hc, _):
      @pl.when(hc + 1 < nhc)
      def _():
        issue(hc + 1)

      drain(hc)

      chunk = idx_v[pl.ds(hc * nh, nl)]
      cvec = chunk % _LANE
      jm = lanes % nh
      bvec = jnp.full((nl,), hc % 2, jnp.int32)
      lo = lanes < nh

      def extract(d, _):
        vals = plsc.load_gather(
            blk_v, [bvec, jm, jnp.full((nl,), d, jnp.int32), cvec])
        plsc.store_compressed(rows_v.at[d, pl.ds(hc * nh, nl)], vals, mask=lo)
        return 0

      lax.fori_loop(0, _EMBED_DIM, extract, 0)
      return 0

    lax.fori_loop(0, nhc, hc_body, 0)
    pltpu.sync_copy(rows_v.at[:, pl.ds(0, per_w)],
                    rows_out.at[:, pl.ds(base, per_w)])

  return k(table_t, u, o, t, i)


def _sc_gather_bias(bias_t, u, o, t, i):
  info = plsc.get_sparse_core_info()
  nw = info.num_cores * info.num_subcores
  nl = info.num_lanes
  n = 4 * _B
  per_w = n // nw
  nchunk = per_w // nl

  mesh = plsc.VectorSubcoreMesh(core_axis_name="c", subcore_axis_name="s")

  @functools.partial(
      pl.kernel,
      out_type=jax.ShapeDtypeStruct((1, n), jnp.float32),
      mesh=mesh,
      compiler_params=pltpu.CompilerParams(needs_layout_passes=False),
      scratch_types=[
          pltpu.VMEM((per_w,), jnp.int32),
          pltpu.VMEM((per_w, _LANE), jnp.float32),
          pltpu.VMEM((1, per_w), jnp.float32),
          pltpu.SemaphoreType.DMA,
      ],
  )
  def k(bias_hbm, u_hbm, o_hbm, t_hbm, i_hbm, bias_out,
        idx_v, blk_v, bias_v, sem_b):
    wid = lax.axis_index("s") * info.num_cores + lax.axis_index("c")
    base = wid * per_w
    _load_my_indices(u_hbm, o_hbm, t_hbm, i_hbm, idx_v, wid, per_w, nl)

    lanes = lax.iota(jnp.int32, nl)

    def issue(j, _):
      chunk = idx_v[pl.ds((j // nl) * nl, nl)]
      idx = jnp.sum(jnp.where(lanes == j % nl, chunk, 0))
      blk = (idx // _LANE) * _LANE
      pltpu.async_copy(bias_hbm.at[:, pl.ds(blk, _LANE)],
                       blk_v.at[pl.ds(j, 1), :], sem_b)
      return 0

    lax.fori_loop(0, per_w, issue, 0)

    def drain(j, _):
      pltpu.make_async_copy(bias_hbm.at[:, pl.ds(0, _LANE)],
                            blk_v.at[pl.ds(0, 1), :], sem_b).wait()
      return 0

    lax.fori_loop(0, per_w, drain, 0)

    def extract(ch, _):
      chunk = idx_v[pl.ds(ch * nl, nl)]
      vals = plsc.load_gather(blk_v, [ch * nl + lanes, chunk % _LANE])
      bias_v[0, pl.ds(ch * nl, nl)] = vals
      return 0

    lax.fori_loop(0, nchunk, extract, 0)
    pltpu.sync_copy(bias_v, bias_out.at[:, pl.ds(base, per_w)])

  return k(bias_t, u, o, t, i)


def _tc_body(rows_ref, biasg_ref, out_ref):
  u = rows_ref[:, 0 * _B:1 * _B]
  o = rows_ref[:, 1 * _B:2 * _B]
  t = rows_ref[:, 2 * _B:3 * _B]
  v = rows_ref[:, 3 * _B:4 * _B]
  s = u + o + t
  cross = jnp.sum(u * o + u * t + o * t, axis=0)
  row_bias = (biasg_ref[0, 0 * _B:1 * _B] + biasg_ref[0, 1 * _B:2 * _B]
              + biasg_ref[0, 2 * _B:3 * _B])
  item_bias = biasg_ref[0, 3 * _B:4 * _B]
  inter = lax.dot_general(
      s, v, dimension_numbers=(((0,), (0,)), ((), ())),
      preferred_element_type=jnp.float32)
  out_ref[...] = inter + (cross + row_bias)[:, None] + item_bias[None, :]


def kernel(user_code, item_code, user_occupation, item_timestamp_rank,
           feature_table, bias_table):
  u = user_code.astype(jnp.int32)
  i = item_code.astype(jnp.int32)
  o = user_occupation.astype(jnp.int32)
  t = item_timestamp_rank.astype(jnp.int32)

  rows_t = _sc_gather_rows(feature_table.T, u, o, t, i)
  bias_g = _sc_gather_bias(bias_table.T, u, o, t, i)

  return pl.pallas_call(
      _tc_body,
      out_shape=jax.ShapeDtypeStruct((_B, _B), jnp.float32),
  )(rows_t, bias_g)


# speedup vs baseline: 1.0580x; 1.0580x over previous
import functools

import jax
import jax.numpy as jnp
from jax import lax
from jax.experimental import pallas as pl
from jax.experimental.pallas import tpu as pltpu
from jax.experimental.pallas import tpu_sc as plsc

_N_USERS = 1000000
_N_ITEMS = 100000
_N_OCC = 1000
_EMBED_DIM = 32
_B = 1024
_LANE = 128

_OFF_O = _N_USERS + _N_ITEMS
_OFF_T = _N_USERS + _N_ITEMS + _N_OCC
_OFF_I = _N_USERS


def _load_my_indices(u_hbm, o_hbm, t_hbm, i_hbm, idx_v, wid, per_w, nl):
  src = wid // 8
  start = (wid % 8) * per_w

  @pl.when(src == 0)
  def _():
    pltpu.sync_copy(u_hbm.at[pl.ds(start, per_w)], idx_v)

  @pl.when(src == 1)
  def _():
    pltpu.sync_copy(o_hbm.at[pl.ds(start, per_w)], idx_v)

  @pl.when(src == 2)
  def _():
    pltpu.sync_copy(t_hbm.at[pl.ds(start, per_w)], idx_v)

  @pl.when(src == 3)
  def _():
    pltpu.sync_copy(i_hbm.at[pl.ds(start, per_w)], idx_v)

  off = (jnp.where(src == 1, _OFF_O, 0)
         + jnp.where(src == 2, _OFF_T, 0)
         + jnp.where(src == 3, _OFF_I, 0)).astype(jnp.int32)

  def add_off(c, _):
    idx_v[pl.ds(c * nl, nl)] = idx_v[pl.ds(c * nl, nl)] + off
    return 0

  lax.fori_loop(0, per_w // nl, add_off, 0)


def _sc_gather(table_t, bias_t, u, o, t, i):
  info = plsc.get_sparse_core_info()
  nw = info.num_cores * info.num_subcores
  nl = info.num_lanes
  n = 4 * _B
  per_w = n // nw
  nchunk = per_w // nl

  mesh = plsc.VectorSubcoreMesh(core_axis_name="c", subcore_axis_name="s")

  nh = nl // 2
  nhc = per_w // nh

  @functools.partial(
      pl.kernel,
      out_type=(
          jax.ShapeDtypeStruct((_EMBED_DIM, n), jnp.float32),
          jax.ShapeDtypeStruct((1, n), jnp.float32),
      ),
      mesh=mesh,
      compiler_params=pltpu.CompilerParams(needs_layout_passes=False),
      scratch_types=[
          pltpu.VMEM((per_w + nh,), jnp.int32),
          pltpu.VMEM((2, nh, _EMBED_DIM, _LANE), jnp.float32),
          pltpu.VMEM((_EMBED_DIM, per_w + nh), jnp.float32),
          pltpu.VMEM((per_w, _LANE), jnp.float32),
          pltpu.VMEM((1, per_w), jnp.float32),
          pltpu.SemaphoreType.DMA,
          pltpu.SemaphoreType.DMA,
          pltpu.SemaphoreType.DMA,
      ],
  )
  def k(table_hbm, bias_hbm, u_hbm, o_hbm, t_hbm, i_hbm, rows_out, bias_out,
        idx_v, blk_v, rows_v, bblk_v, bias_v, sem_0, sem_1, sem_b):
    wid = lax.axis_index("s") * info.num_cores + lax.axis_index("c")
    base = wid * per_w
    _load_my_indices(u_hbm, o_hbm, t_hbm, i_hbm,
                     idx_v.at[pl.ds(0, per_w)], wid, per_w, nl)

    lanes = lax.iota(jnp.int32, nl)

    def bias_issue(j, _):
      chunk = idx_v[pl.ds((j // nl) * nl, nl)]
      idx = jnp.sum(jnp.where(lanes == j % nl, chunk, 0))
      blk = (idx // _LANE) * _LANE
      pltpu.async_copy(bias_hbm.at[:, pl.ds(blk, _LANE)],
                       bblk_v.at[pl.ds(j, 1), :], sem_b)
      return 0

    lax.fori_loop(0, per_w, bias_issue, 0)

    def issue(hc):
      buf = hc % 2
      chunk = idx_v[pl.ds(hc * nh, nl)]

      def one(j, _):
        idx = jnp.sum(jnp.where(lanes == j, chunk, 0))
        blk = pl.multiple_of((idx // _LANE) * _LANE, _LANE)

        @pl.when(buf == 0)
        def _():
          pltpu.async_copy(table_hbm.at[:, pl.ds(blk, _LANE)],
                           blk_v.at[0, j], sem_0)

        @pl.when(buf == 1)
        def _():
          pltpu.async_copy(table_hbm.at[:, pl.ds(blk, _LANE)],
                           blk_v.at[1, j], sem_1)

        return 0

      lax.fori_loop(0, nh, one, 0)

    def drain(hc):
      def one(j, _):
        @pl.when(hc % 2 == 0)
        def _():
          pltpu.make_async_copy(table_hbm.at[:, pl.ds(0, _LANE)],
                                blk_v.at[0, 0], sem_0).wait()

        @pl.when(hc % 2 == 1)
        def _():
          pltpu.make_async_copy(table_hbm.at[:, pl.ds(0, _LANE)],
                                blk_v.at[1, 0], sem_1).wait()

        return 0

      lax.fori_loop(0, nh, one, 0)

    issue(0)

    def hc_body(hc, _):
      @pl.when(hc + 1 < nhc)
      def _():
        issue(hc + 1)

      drain(hc)

      chunk = idx_v[pl.ds(hc * nh, nl)]
      cvec = chunk % _LANE
      jm = lanes % nh
      bvec = jnp.full((nl,), hc % 2, jnp.int32)
      lo = lanes < nh

      def extract(d, _):
        vals = plsc.load_gather(
            blk_v, [bvec, jm, jnp.full((nl,), d, jnp.int32), cvec])
        plsc.store_compressed(rows_v.at[d, pl.ds(hc * nh, nl)], vals, mask=lo)
        return 0

      lax.fori_loop(0, _EMBED_DIM, extract, 0)
      return 0

    lax.fori_loop(0, nhc, hc_body, 0)

    def bias_drain(j, _):
      pltpu.make_async_copy(bias_hbm.at[:, pl.ds(0, _LANE)],
                            bblk_v.at[pl.ds(0, 1), :], sem_b).wait()
      return 0

    lax.fori_loop(0, per_w, bias_drain, 0)

    def bias_extract(ch, _):
      chunk = idx_v[pl.ds(ch * nl, nl)]
      vals = plsc.load_gather(bblk_v, [ch * nl + lanes, chunk % _LANE])
      bias_v[0, pl.ds(ch * nl, nl)] = vals
      return 0

    lax.fori_loop(0, nchunk, bias_extract, 0)
    pltpu.sync_copy(rows_v.at[:, pl.ds(0, per_w)],
                    rows_out.at[:, pl.ds(base, per_w)])
    pltpu.sync_copy(bias_v, bias_out.at[:, pl.ds(base, per_w)])

  return k(table_t, bias_t, u, o, t, i)


def _tc_body(rows_ref, biasg_ref, out_ref):
  u = rows_ref[:, 0 * _B:1 * _B]
  o = rows_ref[:, 1 * _B:2 * _B]
  t = rows_ref[:, 2 * _B:3 * _B]
  v = rows_ref[:, 3 * _B:4 * _B]
  s = u + o + t
  cross = jnp.sum(u * o + u * t + o * t, axis=0)
  row_bias = (biasg_ref[0, 0 * _B:1 * _B] + biasg_ref[0, 1 * _B:2 * _B]
              + biasg_ref[0, 2 * _B:3 * _B])
  item_bias = biasg_ref[0, 3 * _B:4 * _B]
  inter = lax.dot_general(
      s, v, dimension_numbers=(((0,), (0,)), ((), ())),
      preferred_element_type=jnp.float32)
  out_ref[...] = inter + (cross + row_bias)[:, None] + item_bias[None, :]


def kernel(user_code, item_code, user_occupation, item_timestamp_rank,
           feature_table, bias_table):
  u = user_code.astype(jnp.int32)
  i = item_code.astype(jnp.int32)
  o = user_occupation.astype(jnp.int32)
  t = item_timestamp_rank.astype(jnp.int32)

  rows_t, bias_g = _sc_gather(feature_table.T, bias_table.T, u, o, t, i)

  return pl.pallas_call(
      _tc_body,
      out_shape=jax.ShapeDtypeStruct((_B, _B), jnp.float32),
  )(rows_t, bias_g)


# speedup vs baseline: 1.1039x; 1.0434x over previous
import functools

import jax
import jax.numpy as jnp
from jax import lax
from jax.experimental import pallas as pl
from jax.experimental.pallas import tpu as pltpu
from jax.experimental.pallas import tpu_sc as plsc

_N_USERS = 1000000
_N_ITEMS = 100000
_N_OCC = 1000
_EMBED_DIM = 32
_B = 1024
_LANE = 128

_OFF_O = _N_USERS + _N_ITEMS
_OFF_T = _N_USERS + _N_ITEMS + _N_OCC
_OFF_I = _N_USERS


def _load_my_indices(u_hbm, o_hbm, t_hbm, i_hbm, idx_v, wid, per_w, nl):
  src = wid // 8
  start = (wid % 8) * per_w

  @pl.when(src == 0)
  def _():
    pltpu.sync_copy(u_hbm.at[pl.ds(start, per_w)], idx_v)

  @pl.when(src == 1)
  def _():
    pltpu.sync_copy(o_hbm.at[pl.ds(start, per_w)], idx_v)

  @pl.when(src == 2)
  def _():
    pltpu.sync_copy(t_hbm.at[pl.ds(start, per_w)], idx_v)

  @pl.when(src == 3)
  def _():
    pltpu.sync_copy(i_hbm.at[pl.ds(start, per_w)], idx_v)

  off = (jnp.where(src == 1, _OFF_O, 0)
         + jnp.where(src == 2, _OFF_T, 0)
         + jnp.where(src == 3, _OFF_I, 0)).astype(jnp.int32)

  def add_off(c, _):
    idx_v[pl.ds(c * nl, nl)] = idx_v[pl.ds(c * nl, nl)] + off
    return 0

  lax.fori_loop(0, per_w // nl, add_off, 0)


def _sc_gather(table_t, bias_t, u, o, t, i):
  info = plsc.get_sparse_core_info()
  nw = info.num_cores * info.num_subcores
  nl = info.num_lanes
  n = 4 * _B
  per_w = n // nw
  nchunk = per_w // nl

  mesh = plsc.VectorSubcoreMesh(core_axis_name="c", subcore_axis_name="s")

  nh = nl // 2
  nhc = per_w // nh

  @functools.partial(
      pl.kernel,
      out_type=(
          jax.ShapeDtypeStruct((_EMBED_DIM, n), jnp.float32),
          jax.ShapeDtypeStruct((1, n), jnp.float32),
      ),
      mesh=mesh,
      compiler_params=pltpu.CompilerParams(needs_layout_passes=False),
      scratch_types=[
          pltpu.VMEM((per_w + nh,), jnp.int32),
          pltpu.VMEM((2, nh, _EMBED_DIM, _LANE), jnp.float32),
          pltpu.VMEM((_EMBED_DIM, per_w + nh), jnp.float32),
          pltpu.VMEM((per_w, _LANE), jnp.float32),
          pltpu.VMEM((1, per_w), jnp.float32),
          pltpu.SemaphoreType.DMA,
          pltpu.SemaphoreType.DMA,
          pltpu.SemaphoreType.DMA,
      ],
  )
  def k(table_hbm, bias_hbm, u_hbm, o_hbm, t_hbm, i_hbm, rows_out, bias_out,
        idx_v, blk_v, rows_v, bblk_v, bias_v, sem_0, sem_1, sem_b):
    wid = lax.axis_index("s") * info.num_cores + lax.axis_index("c")
    base = wid * per_w
    _load_my_indices(u_hbm, o_hbm, t_hbm, i_hbm,
                     idx_v.at[pl.ds(0, per_w)], wid, per_w, nl)

    lanes = lax.iota(jnp.int32, nl)

    def issue(hc):
      buf = hc % 2
      chunk = idx_v[pl.ds(hc * nh, nl)]

      def one(j, _):
        idx = jnp.sum(jnp.where(lanes == j, chunk, 0))
        blk = pl.multiple_of((idx // _LANE) * _LANE, _LANE)
        pltpu.async_copy(bias_hbm.at[:, pl.ds(blk, _LANE)],
                         bblk_v.at[pl.ds(hc * nh + j, 1), :], sem_b)

        @pl.when(buf == 0)
        def _():
          pltpu.async_copy(table_hbm.at[:, pl.ds(blk, _LANE)],
                           blk_v.at[0, j], sem_0)

        @pl.when(buf == 1)
        def _():
          pltpu.async_copy(table_hbm.at[:, pl.ds(blk, _LANE)],
                           blk_v.at[1, j], sem_1)

        return 0

      lax.fori_loop(0, nh, one, 0)

    def drain(hc):
      def one(j, _):
        @pl.when(hc % 2 == 0)
        def _():
          pltpu.make_async_copy(table_hbm.at[:, pl.ds(0, _LANE)],
                                blk_v.at[0, 0], sem_0).wait()

        @pl.when(hc % 2 == 1)
        def _():
          pltpu.make_async_copy(table_hbm.at[:, pl.ds(0, _LANE)],
                                blk_v.at[1, 0], sem_1).wait()

        return 0

      lax.fori_loop(0, nh, one, 0)

    issue(0)

    def hc_body(hc, _):
      @pl.when(hc + 1 < nhc)
      def _():
        issue(hc + 1)

      drain(hc)

      chunk = idx_v[pl.ds(hc * nh, nl)]
      cvec = chunk % _LANE
      jm = lanes % nh
      bvec = jnp.full((nl,), hc % 2, jnp.int32)
      lo = lanes < nh
      hi = jnp.logical_not(lo)
      dsel = (lanes // nh) * (_EMBED_DIM // 2)

      def extract(d, _):
        vals = plsc.load_gather(blk_v, [bvec, jm, dsel + d, cvec])
        plsc.store_compressed(rows_v.at[d, pl.ds(hc * nh, nl)], vals, mask=lo)
        plsc.store_compressed(
            rows_v.at[d + _EMBED_DIM // 2, pl.ds(hc * nh, nl)], vals, mask=hi)
        return 0

      lax.fori_loop(0, _EMBED_DIM // 2, extract, 0)
      return 0

    lax.fori_loop(0, nhc, hc_body, 0)

    def bias_drain(j, _):
      pltpu.make_async_copy(bias_hbm.at[:, pl.ds(0, _LANE)],
                            bblk_v.at[pl.ds(0, 1), :], sem_b).wait()
      return 0

    lax.fori_loop(0, per_w, bias_drain, 0)

    def bias_extract(ch, _):
      chunk = idx_v[pl.ds(ch * nl, nl)]
      vals = plsc.load_gather(bblk_v, [ch * nl + lanes, chunk % _LANE])
      bias_v[0, pl.ds(ch * nl, nl)] = vals
      return 0

    lax.fori_loop(0, nchunk, bias_extract, 0)
    pltpu.sync_copy(rows_v.at[:, pl.ds(0, per_w)],
                    rows_out.at[:, pl.ds(base, per_w)])
    pltpu.sync_copy(bias_v, bias_out.at[:, pl.ds(base, per_w)])

  return k(table_t, bias_t, u, o, t, i)


def _tc_body(rows_ref, biasg_ref, out_ref):
  u = rows_ref[:, 0 * _B:1 * _B]
  o = rows_ref[:, 1 * _B:2 * _B]
  t = rows_ref[:, 2 * _B:3 * _B]
  v = rows_ref[:, 3 * _B:4 * _B]
  s = u + o + t
  cross = jnp.sum(u * o + u * t + o * t, axis=0)
  row_bias = (biasg_ref[0, 0 * _B:1 * _B] + biasg_ref[0, 1 * _B:2 * _B]
              + biasg_ref[0, 2 * _B:3 * _B])
  item_bias = biasg_ref[0, 3 * _B:4 * _B]
  inter = lax.dot_general(
      s, v, dimension_numbers=(((0,), (0,)), ((), ())),
      preferred_element_type=jnp.float32)
  out_ref[...] = inter + (cross + row_bias)[:, None] + item_bias[None, :]


def kernel(user_code, item_code, user_occupation, item_timestamp_rank,
           feature_table, bias_table):
  u = user_code.astype(jnp.int32)
  i = item_code.astype(jnp.int32)
  o = user_occupation.astype(jnp.int32)
  t = item_timestamp_rank.astype(jnp.int32)

  rows_t, bias_g = _sc_gather(feature_table.T, bias_table.T, u, o, t, i)

  return pl.pallas_call(
      _tc_body,
      out_shape=jax.ShapeDtypeStruct((_B, _B), jnp.float32),
  )(rows_t, bias_g)


# speedup vs baseline: 1.1042x; 1.0002x over previous
import functools

import jax
import jax.numpy as jnp
from jax import lax
from jax.experimental import pallas as pl
from jax.experimental.pallas import tpu as pltpu
from jax.experimental.pallas import tpu_sc as plsc

_N_USERS = 1000000
_N_ITEMS = 100000
_N_OCC = 1000
_EMBED_DIM = 32
_B = 1024
_LANE = 128

_OFF_O = _N_USERS + _N_ITEMS
_OFF_T = _N_USERS + _N_ITEMS + _N_OCC
_OFF_I = _N_USERS


def _load_my_indices(u_hbm, o_hbm, t_hbm, i_hbm, idx_v, wid, per_w, nl):
  src = wid // 8
  start = (wid % 8) * per_w

  @pl.when(src == 0)
  def _():
    pltpu.sync_copy(u_hbm.at[pl.ds(start, per_w)], idx_v)

  @pl.when(src == 1)
  def _():
    pltpu.sync_copy(o_hbm.at[pl.ds(start, per_w)], idx_v)

  @pl.when(src == 2)
  def _():
    pltpu.sync_copy(t_hbm.at[pl.ds(start, per_w)], idx_v)

  @pl.when(src == 3)
  def _():
    pltpu.sync_copy(i_hbm.at[pl.ds(start, per_w)], idx_v)

  off = (jnp.where(src == 1, _OFF_O, 0)
         + jnp.where(src == 2, _OFF_T, 0)
         + jnp.where(src == 3, _OFF_I, 0)).astype(jnp.int32)

  def add_off(c, _):
    idx_v[pl.ds(c * nl, nl)] = idx_v[pl.ds(c * nl, nl)] + off
    return 0

  lax.fori_loop(0, per_w // nl, add_off, 0)


def _sc_gather(table_t, bias_t, u, o, t, i):
  info = plsc.get_sparse_core_info()
  nw = info.num_cores * info.num_subcores
  nl = info.num_lanes
  n = 4 * _B
  per_w = n // nw
  nchunk = per_w // nl

  mesh = plsc.VectorSubcoreMesh(core_axis_name="c", subcore_axis_name="s")

  nh = nl // 2
  nhc = per_w // nh

  @functools.partial(
      pl.kernel,
      out_type=(
          jax.ShapeDtypeStruct((_EMBED_DIM, n), jnp.float32),
          jax.ShapeDtypeStruct((1, n), jnp.float32),
      ),
      mesh=mesh,
      compiler_params=pltpu.CompilerParams(needs_layout_passes=False),
      scratch_types=[
          pltpu.VMEM((per_w + nh,), jnp.int32),
          pltpu.VMEM((2, nh, _EMBED_DIM, _LANE), jnp.float32),
          pltpu.VMEM((_EMBED_DIM, per_w + nh), jnp.float32),
          pltpu.VMEM((per_w, _LANE), jnp.float32),
          pltpu.VMEM((1, per_w), jnp.float32),
          pltpu.SemaphoreType.DMA,
          pltpu.SemaphoreType.DMA,
          pltpu.SemaphoreType.DMA,
      ],
  )
  def k(table_hbm, bias_hbm, u_hbm, o_hbm, t_hbm, i_hbm, rows_out, bias_out,
        idx_v, blk_v, rows_v, bblk_v, bias_v, sem_0, sem_1, sem_b):
    wid = lax.axis_index("s") * info.num_cores + lax.axis_index("c")
    base = wid * per_w
    _load_my_indices(u_hbm, o_hbm, t_hbm, i_hbm,
                     idx_v.at[pl.ds(0, per_w)], wid, per_w, nl)

    lanes = lax.iota(jnp.int32, nl)

    def issue(hc):
      buf = hc % 2
      chunk = idx_v[pl.ds(hc * nh, nl)]

      def one(j, _):
        idx = jnp.sum(jnp.where(lanes == j, chunk, 0))
        blk = pl.multiple_of((idx // _LANE) * _LANE, _LANE)
        pltpu.async_copy(bias_hbm.at[:, pl.ds(blk, _LANE)],
                         bblk_v.at[pl.ds(hc * nh + j, 1), :], sem_b)

        @pl.when(buf == 0)
        def _():
          pltpu.async_copy(table_hbm.at[:, pl.ds(blk, _LANE)],
                           blk_v.at[0, j], sem_0)

        @pl.when(buf == 1)
        def _():
          pltpu.async_copy(table_hbm.at[:, pl.ds(blk, _LANE)],
                           blk_v.at[1, j], sem_1)

        return 0

      lax.fori_loop(0, nh, one, 0)

    def drain(hc):
      def one(j, _):
        @pl.when(hc % 2 == 0)
        def _():
          pltpu.make_async_copy(table_hbm.at[:, pl.ds(0, _LANE)],
                                blk_v.at[0, 0], sem_0).wait()

        @pl.when(hc % 2 == 1)
        def _():
          pltpu.make_async_copy(table_hbm.at[:, pl.ds(0, _LANE)],
                                blk_v.at[1, 0], sem_1).wait()

        return 0

      lax.fori_loop(0, nh, one, 0)

    issue(0)

    def hc_body(hc, _):
      @pl.when(hc + 1 < nhc)
      def _():
        issue(hc + 1)

      drain(hc)

      jm = lanes % nh
      cvec = plsc.load_gather(idx_v, [hc * nh + jm]) % _LANE
      bvec = jnp.full((nl,), hc % 2, jnp.int32)
      lo = lanes < nh
      hi = jnp.logical_not(lo)
      dsel = (lanes // nh) * (_EMBED_DIM // 2)

      def extract(d, _):
        vals = plsc.load_gather(blk_v, [bvec, jm, dsel + d, cvec])
        plsc.store_compressed(rows_v.at[d, pl.ds(hc * nh, nl)], vals, mask=lo)
        plsc.store_compressed(
            rows_v.at[d + _EMBED_DIM // 2, pl.ds(hc * nh, nl)], vals, mask=hi)
        return 0

      lax.fori_loop(0, _EMBED_DIM // 2, extract, 0)
      return 0

    lax.fori_loop(0, nhc, hc_body, 0)

    def bias_drain(j, _):
      pltpu.make_async_copy(bias_hbm.at[:, pl.ds(0, _LANE)],
                            bblk_v.at[pl.ds(0, 1), :], sem_b).wait()
      return 0

    lax.fori_loop(0, per_w, bias_drain, 0)

    def bias_extract(ch, _):
      chunk = idx_v[pl.ds(ch * nl, nl)]
      vals = plsc.load_gather(bblk_v, [ch * nl + lanes, chunk % _LANE])
      bias_v[0, pl.ds(ch * nl, nl)] = vals
      return 0

    lax.fori_loop(0, nchunk, bias_extract, 0)
    pltpu.sync_copy(rows_v.at[:, pl.ds(0, per_w)],
                    rows_out.at[:, pl.ds(base, per_w)])
    pltpu.sync_copy(bias_v, bias_out.at[:, pl.ds(base, per_w)])

  return k(table_t, bias_t, u, o, t, i)


def _tc_body(rows_ref, biasg_ref, out_ref):
  u = rows_ref[:, 0 * _B:1 * _B]
  o = rows_ref[:, 1 * _B:2 * _B]
  t = rows_ref[:, 2 * _B:3 * _B]
  v = rows_ref[:, 3 * _B:4 * _B]
  s = u + o + t
  cross = jnp.sum(u * o + u * t + o * t, axis=0)
  row_bias = (biasg_ref[0, 0 * _B:1 * _B] + biasg_ref[0, 1 * _B:2 * _B]
              + biasg_ref[0, 2 * _B:3 * _B])
  item_bias = biasg_ref[0, 3 * _B:4 * _B]
  inter = lax.dot_general(
      s, v, dimension_numbers=(((0,), (0,)), ((), ())),
      preferred_element_type=jnp.float32)
  out_ref[...] = inter + (cross + row_bias)[:, None] + item_bias[None, :]


def kernel(user_code, item_code, user_occupation, item_timestamp_rank,
           feature_table, bias_table):
  u = user_code.astype(jnp.int32)
  i = item_code.astype(jnp.int32)
  o = user_occupation.astype(jnp.int32)
  t = item_timestamp_rank.astype(jnp.int32)

  rows_t, bias_g = _sc_gather(feature_table.T, bias_table.T, u, o, t, i)

  return pl.pallas_call(
      _tc_body,
      out_shape=jax.ShapeDtypeStruct((_B, _B), jnp.float32),
  )(rows_t, bias_g)
